# Initial kernel scaffold; baseline (speedup 1.0000x reference)
import jax
import jax.numpy as jnp
from jax.experimental import pallas as pl


def kernel(x, emb_weight):
    raise NotImplementedError("write your pallas kernel here")


# speedup vs baseline: 1.0729x; 1.0729x over previous
import functools

import jax
import jax.numpy as jnp
from jax import lax
from jax.experimental import pallas as pl
from jax.experimental.pallas import tpu as pltpu
from jax.experimental.pallas import tpu_sc as plsc

_B = 4096
_T = 200
_C = 16
_NW = 32
_BPW = _B // _NW
_G = 4
_NG = _BPW // _G
_ROWS = _G * _T
_CHUNK = 128


def _sc_kernel(x_hbm, tbl_hbm, out_hbm, idx_v, rows0, rows1, ot0, ot1,
               gsem, osem):
    cid = lax.axis_index("c")
    sid = lax.axis_index("s")
    wid = sid * 2 + cid
    ibase = wid * (_BPW * _T)
    obase = wid * (_BPW * _C * _T)

    pltpu.sync_copy(x_hbm.at[pl.ds(ibase, _BPW * _T)], idx_v)

    rows_bufs = (rows0, rows1)
    out_bufs = (ot0, ot1)

    def start_gathers(g, buf):
        handles = []
        base = g * _ROWS
        off = 0
        while off < _ROWS:
            n = min(_CHUNK, _ROWS - off)
            h = pltpu.async_copy(
                tbl_hbm.at[idx_v.at[pl.ds(base + off, n)]],
                rows_bufs[buf].at[pl.ds(off, n)],
                gsem,
            )
            handles.append(h)
            off += n
        return handles

    ids200 = lax.iota(jnp.int32, 16) * _T

    pending = {0: start_gathers(0, 0)}
    out_pending = {}
    for g in range(_NG):
        buf = g & 1
        for h in pending.pop(g):
            h.wait()
        if g + 1 < _NG:
            pending[g + 1] = start_gathers(g + 1, (g + 1) & 1)
        if g - 2 in out_pending:
            out_pending.pop(g - 2).wait()

        rv = rows_bufs[buf]
        ov = out_bufs[buf]
        for bb in range(_G):
            base_ids = ids200 + jnp.int32(bb * _C * _T)

            def body(t, _, bb=bb, base_ids=base_ids, rv=rv, ov=ov):
                row = rv[bb * _T + t]
                plsc.store_scatter(ov, [base_ids + t], row)
                return 0

            lax.fori_loop(0, _T, body, 0)

        out_pending[g] = pltpu.async_copy(
            ov.at[:],
            out_hbm.at[pl.ds(obase + g * (_G * _C * _T), _G * _C * _T)],
            osem,
        )
    for h in out_pending.values():
        h.wait()


@jax.jit
def kernel(x, emb_weight):
    mesh = plsc.VectorSubcoreMesh(core_axis_name="c", subcore_axis_name="s")
    run = functools.partial(
        pl.kernel,
        mesh=mesh,
        compiler_params=pltpu.CompilerParams(
            needs_layout_passes=False, use_tc_tiling_on_sc=False
        ),
        out_type=jax.ShapeDtypeStruct((_B * _C * _T,), jnp.float32),
        scratch_types=[
            pltpu.VMEM((_BPW * _T,), jnp.int32),
            pltpu.VMEM((_ROWS, _C), jnp.float32),
            pltpu.VMEM((_ROWS, _C), jnp.float32),
            pltpu.VMEM((_G * _C * _T,), jnp.float32),
            pltpu.VMEM((_G * _C * _T,), jnp.float32),
            pltpu.SemaphoreType.DMA,
            pltpu.SemaphoreType.DMA,
        ],
    )(_sc_kernel)
    x_flat = x.astype(jnp.int32).reshape(_B * _T)
    out = run(x_flat, emb_weight)
    return out.reshape(_B, _C, _T)


# speedup vs baseline: 1.7289x; 1.6115x over previous
import functools

import jax
import jax.numpy as jnp
from jax import lax
from jax.experimental import pallas as pl
from jax.experimental.pallas import tpu as pltpu
from jax.experimental.pallas import tpu_sc as plsc

_B = 4096
_T = 200
_C = 16
_V = 1000000
_NW = 32
_BPW = _B // _NW
_G = 4
_NG = _BPW // _G
_ROWS = _G * _T
_CHUNK = 128

_LT = (_V + 127) // 128
_TAIL = (_V - (_LT - 1) * 128) * _C


def _transpose_kernel(tbl_t_hbm, out_hbm, bufs, cb0, cb1, lsems, ws0, ws1):
    cid = lax.axis_index("c")
    sid = lax.axis_index("s")
    wid = sid * 2 + cid
    n_t = jnp.where(wid < 5, 245, 244)
    t0 = jnp.where(wid < 5, wid * 245, 1225 + (wid - 5) * 244)
    ids16 = lax.iota(jnp.int32, 16) * _C
    cbs = (cb0, cb1)
    wsems = (ws0, ws1)

    def load(j, slot):
        pltpu.make_async_copy(
            tbl_t_hbm.at[:, pl.ds(j * 128, 128)], bufs.at[slot], lsems.at[slot]
        ).start()

    def wait_load(slot):
        pltpu.make_async_copy(
            tbl_t_hbm.at[:, pl.ds(0, 128)], bufs.at[slot], lsems.at[slot]
        ).wait()

    def wait_write(p, n=2048):
        pltpu.make_async_copy(
            cbs[p].at[pl.ds(0, n)], out_hbm.at[pl.ds(0, n)], wsems[p]
        ).wait()

    def scatter_tile(slot, cb):
        buf = bufs.at[slot]
        for c in range(_C):
            for l0 in range(0, 128, 16):
                v = buf[c, pl.ds(l0, 16)]
                plsc.store_scatter(cb, [ids16 + (l0 * _C + c)], v)

    def write_tile(j, p):
        @pl.when(j == _LT - 1)
        def _():
            pltpu.make_async_copy(
                cbs[p].at[pl.ds(0, _TAIL)],
                out_hbm.at[pl.ds(j * 2048, _TAIL)], wsems[p]
            ).start()

        @pl.when(j != _LT - 1)
        def _():
            pltpu.make_async_copy(
                cbs[p].at[:], out_hbm.at[pl.ds(j * 2048, 2048)], wsems[p]
            ).start()

    for q in range(4):
        load(t0 + q, q)

    def body(k, carry):
        for q in range(4):
            m = k * 4 + q
            j = t0 + m
            wait_load(q)

            @pl.when(m >= 2)
            def _():
                wait_write(q & 1)

            scatter_tile(q, cbs[q & 1])

            @pl.when(m + 4 < n_t)
            def _():
                load(j + 4, q)

            write_tile(j, q & 1)
        return carry

    lax.fori_loop(0, 61, body, 0)

    @pl.when(n_t > 244)
    def _():
        wait_load(0)
        wait_write(0)
        scatter_tile(0, cb0)
        write_tile(t0 + 244, 0)

    wait_write(0)

    @pl.when(wid == 31)
    def _():
        wait_write(1, _TAIL)

    @pl.when(wid != 31)
    def _():
        wait_write(1)


def _gather_kernel(x_hbm, tbl_hbm, out_hbm, idx_v, rows0, rows1, ot0, ot1,
                   gsem, osem):
    cid = lax.axis_index("c")
    sid = lax.axis_index("s")
    wid = sid * 2 + cid
    ibase = wid * (_BPW * _T)
    obase = wid * (_BPW * _C * _T)

    pltpu.sync_copy(x_hbm.at[pl.ds(ibase, _BPW * _T)], idx_v)

    rows_bufs = (rows0, rows1)
    out_bufs = (ot0, ot1)

    def start_gathers(g, buf):
        handles = []
        base = g * _ROWS
        off = 0
        while off < _ROWS:
            n = min(_CHUNK, _ROWS - off)
            h = pltpu.async_copy(
                tbl_hbm.at[idx_v.at[pl.ds(base + off, n)]],
                rows_bufs[buf].at[pl.ds(off, n)],
                gsem,
            )
            handles.append(h)
            off += n
        return handles

    ids200 = lax.iota(jnp.int32, 16) * _T

    pending = {0: start_gathers(0, 0)}
    out_pending = {}
    for g in range(_NG):
        buf = g & 1
        for h in pending.pop(g):
            h.wait()
        if g + 1 < _NG:
            pending[g + 1] = start_gathers(g + 1, (g + 1) & 1)
        if g - 2 in out_pending:
            out_pending.pop(g - 2).wait()

        rv = rows_bufs[buf]
        ov = out_bufs[buf]
        for bb in range(_G):
            base_ids = ids200 + jnp.int32(bb * _C * _T)

            def body(t, _, bb=bb, base_ids=base_ids, rv=rv, ov=ov):
                row = rv[bb * _T + t]
                plsc.store_scatter(ov, [base_ids + t], row)
                return 0

            lax.fori_loop(0, _T, body, 0)

        out_pending[g] = pltpu.async_copy(
            ov.at[:],
            out_hbm.at[pl.ds(obase + g * (_G * _C * _T), _G * _C * _T)],
            osem,
        )
    for h in out_pending.values():
        h.wait()


@jax.jit
def kernel(x, emb_weight):
    mesh = plsc.VectorSubcoreMesh(core_axis_name="c", subcore_axis_name="s")

    relayout = functools.partial(
        pl.kernel,
        mesh=mesh,
        compiler_params=pltpu.CompilerParams(
            needs_layout_passes=False, use_tc_tiling_on_sc=True
        ),
        out_type=jax.ShapeDtypeStruct((_V * _C,), jnp.float32),
        scratch_types=[
            pltpu.VMEM((4, _C, 128), jnp.float32),
            pltpu.VMEM((2048,), jnp.float32),
            pltpu.VMEM((2048,), jnp.float32),
            pltpu.SemaphoreType.DMA((4,)),
            pltpu.SemaphoreType.DMA,
            pltpu.SemaphoreType.DMA,
        ],
    )(_transpose_kernel)

    gather = functools.partial(
        pl.kernel,
        mesh=mesh,
        compiler_params=pltpu.CompilerParams(
            needs_layout_passes=False, use_tc_tiling_on_sc=False
        ),
        out_type=jax.ShapeDtypeStruct((_B * _C * _T,), jnp.float32),
        scratch_types=[
            pltpu.VMEM((_BPW * _T,), jnp.int32),
            pltpu.VMEM((_ROWS, _C), jnp.float32),
            pltpu.VMEM((_ROWS, _C), jnp.float32),
            pltpu.VMEM((_G * _C * _T,), jnp.float32),
            pltpu.VMEM((_G * _C * _T,), jnp.float32),
            pltpu.SemaphoreType.DMA,
            pltpu.SemaphoreType.DMA,
        ],
    )(_gather_kernel)

    tbl_flat = relayout(emb_weight.T)
    tbl = tbl_flat.reshape(_V, _C)
    x_flat = x.astype(jnp.int32).reshape(_B * _T)
    out = gather(x_flat, tbl)
    return out.reshape(_B, _C, _T)
